# baseline (speedup 1.0000x reference)
import functools

import jax
import jax.numpy as jnp
from jax import lax
from jax.experimental import pallas as pl
from jax.experimental.pallas import tpu as pltpu
from jax.experimental.pallas import tpu_sc as plsc

N_NODES = 10000
N_EDGES = 160000
TW = 25
H = 128
NLAYERS = 6

N_PAD = 10240
E_PAD = 163840
NC = 2
NS = 16
NW = NC * NS
PER_W = E_PAD // NW
ROWS_W = PER_W // 128
NCH = ROWS_W // 2
NCH2 = ROWS_W
ROWS_HA = 24
ROWS_HB = 16
E2A = ROWS_HA * NW * 128
E2B = ROWS_HB * NW * 128
TILE_ROWS = N_PAD // NS

NBLK = 1024
EBLK = 1024

f32 = jnp.float32
i32 = jnp.int32


def _swish(x):
    return x * jax.nn.sigmoid(x)


def _full(shape):
    return pl.BlockSpec(shape, lambda i: tuple(0 for _ in shape))


def _rows(blk, width, off=0):
    return pl.BlockSpec((blk, width), lambda i, o=off: (i + o, 0))


def _enc_ab_body(u, e1T, e1b, e2T, e2b, waT, wuT, b1, wbT, h_o, a_o, b_o):
    uu = u[...]
    h1 = _swish(jnp.dot(uu, e1T[...], preferred_element_type=f32) + e1b[...])
    h = _swish(jnp.dot(h1, e2T[...], preferred_element_type=f32) + e2b[...])
    uw = jnp.dot(uu, wuT[...], preferred_element_type=f32)
    h_o[...] = h
    a_o[...] = jnp.dot(h, waT[...], preferred_element_type=f32) + uw + b1[...]
    b_o[...] = jnp.dot(h, wbT[...], preferred_element_type=f32) - uw


def _enc_ab(u, e1T, e1b, e2T, e2b, waT, wuT, b1, wbT):
    return pl.pallas_call(
        _enc_ab_body,
        grid=(N_PAD // NBLK,),
        in_specs=[_rows(NBLK, TW), _full((TW, H)), _full((1, H)),
                  _full((H, H)), _full((1, H)), _full((H, H)),
                  _full((TW, H)), _full((1, H)), _full((H, H))],
        out_specs=[_rows(NBLK, H)] * 3,
        out_shape=[jax.ShapeDtypeStruct((N_PAD, H), f32)] * 3,
    )(u, e1T, e1b, e2T, e2b, waT, wuT, b1, wbT)


def _edge_body(ga, gb, dist, wd, m2T, m2b, m_o):
    pre = ga[...] + gb[...] + dist[...] * wd[...]
    m1 = _swish(pre)
    m_o[...] = _swish(jnp.dot(m1, m2T[...], preferred_element_type=f32) + m2b[...])


def _edge(gA, gB, dist2, wd, m2T, m2b):
    eh = gA.shape[0]
    return pl.pallas_call(
        _edge_body,
        grid=(eh // EBLK,),
        in_specs=[_rows(EBLK, H), _rows(EBLK, H), _rows(EBLK, 1),
                  _full((1, H)), _full((H, H)), _full((1, H))],
        out_specs=_rows(EBLK, H),
        out_shape=jax.ShapeDtypeStruct((eh, H), f32),
    )(gA, gB, dist2, wd, m2T, m2b)


def _upd_ab_body(h, p0, p1, q0, q1, c0, c1, u, u1hT, u1aT, u1b, u2T, u2b,
                 waT, wuT, b1, wbT, hn_o, a_o, b_o):
    hh = h[...]
    cnt = jnp.sum(c0[...] + c1[...], axis=1, keepdims=True)
    agg = (p0[...] + p1[...] + q0[...] + q1[...]) / jnp.maximum(cnt, 1.0)
    t = _swish(jnp.dot(hh, u1hT[...], preferred_element_type=f32)
               + jnp.dot(agg, u1aT[...], preferred_element_type=f32) + u1b[...])
    upd = _swish(jnp.dot(t, u2T[...], preferred_element_type=f32) + u2b[...])
    hn = hh + upd
    hn_o[...] = hn
    uw = jnp.dot(u[...], wuT[...], preferred_element_type=f32)
    a_o[...] = jnp.dot(hn, waT[...], preferred_element_type=f32) + uw + b1[...]
    b_o[...] = jnp.dot(hn, wbT[...], preferred_element_type=f32) - uw


def _upd_ab(h, P1, P2, C, u, u1hT, u1aT, u1b, u2T, u2b, waT, wuT, b1, wbT):
    nb = N_PAD // NBLK
    return pl.pallas_call(
        _upd_ab_body,
        grid=(nb,),
        in_specs=[_rows(NBLK, H), _rows(NBLK, H), _rows(NBLK, H, off=nb),
                  _rows(NBLK, H), _rows(NBLK, H, off=nb),
                  _rows(NBLK, H), _rows(NBLK, H, off=nb), _rows(NBLK, TW),
                  _full((H, H)), _full((H, H)), _full((1, H)),
                  _full((H, H)), _full((1, H)),
                  _full((H, H)), _full((TW, H)), _full((1, H)), _full((H, H))],
        out_specs=[_rows(NBLK, H)] * 3,
        out_shape=[jax.ShapeDtypeStruct((N_PAD, H), f32)] * 3,
    )(h, P1, P1, P2, P2, C, C, u, u1hT, u1aT, u1b, u2T, u2b, waT, wuT, b1, wbT)


def _upd_last_body(h, p0, p1, q0, q1, c0, c1, u1hT, u1aT, u1b, u2T, u2b, hn_o):
    hh = h[...]
    cnt = jnp.sum(c0[...] + c1[...], axis=1, keepdims=True)
    agg = (p0[...] + p1[...] + q0[...] + q1[...]) / jnp.maximum(cnt, 1.0)
    t = _swish(jnp.dot(hh, u1hT[...], preferred_element_type=f32)
               + jnp.dot(agg, u1aT[...], preferred_element_type=f32) + u1b[...])
    upd = _swish(jnp.dot(t, u2T[...], preferred_element_type=f32) + u2b[...])
    hn_o[...] = hh + upd


def _upd_last(h, P1, P2, C, u1hT, u1aT, u1b, u2T, u2b):
    nb = N_PAD // NBLK
    return pl.pallas_call(
        _upd_last_body,
        grid=(nb,),
        in_specs=[_rows(NBLK, H), _rows(NBLK, H), _rows(NBLK, H, off=nb),
                  _rows(NBLK, H), _rows(NBLK, H, off=nb),
                  _rows(NBLK, H), _rows(NBLK, H, off=nb),
                  _full((H, H)), _full((H, H)), _full((1, H)),
                  _full((H, H)), _full((1, H))],
        out_specs=_rows(NBLK, H),
        out_shape=jax.ShapeDtypeStruct((N_PAD, H), f32),
    )(h, P1, P1, P2, P2, C, C, u1hT, u1aT, u1b, u2T, u2b)


def _dec_body(h, u, M1, b1r, M2, b2r, o_ref):
    y = _swish(jnp.dot(h[...], M1[...], preferred_element_type=f32) + b1r[...])
    z = jnp.dot(y, M2[...], preferred_element_type=f32) + b2r[...]
    dtv = (lax.broadcasted_iota(i32, (NBLK, TW), 1) + 1).astype(f32)
    o_ref[...] = u[...][:, TW - 1:TW] + dtv * z


def _dec(h, u, M1, b1r, M2, b2r):
    return pl.pallas_call(
        _dec_body,
        grid=(N_PAD // NBLK,),
        in_specs=[_rows(NBLK, H), _rows(NBLK, TW), _full((H, 8 * 38)),
                  _full((1, 8 * 38)), _full((8 * 38, TW)), _full((1, TW))],
        out_specs=_rows(NBLK, TW),
        out_shape=jax.ShapeDtypeStruct((N_PAD, TW), f32),
    )(h, u, M1, b1r, M2, b2r)


def _sc_mesh():
    return plsc.VectorSubcoreMesh(core_axis_name="c", subcore_axis_name="s")


def _gather_ab(tabA, tabB, dst2, src2, half):
    RING = 3
    NCHW = ROWS_HA if half == 0 else ROWS_HB
    EH = NCHW * NW * 128

    @functools.partial(
        pl.kernel,
        out_type=(jax.ShapeDtypeStruct((EH, H), f32),
                  jax.ShapeDtypeStruct((EH, H), f32)),
        mesh=_sc_mesh(),
        scratch_types=[
            pltpu.VMEM((NCHW, 128), i32), pltpu.VMEM((NCHW, 128), i32),
            pltpu.VMEM((128, H), f32), pltpu.VMEM((128, H), f32),
        ],
    )
    def k(tabA_h, tabB_h, dst_h, src_h, outA_h, outB_h,
          idxd, idxs, bufA, bufB):
        c = lax.axis_index("c")
        s = lax.axis_index("s")
        wid = s * NC + c
        row0 = wid * ROWS_W + half * ROWS_HA
        e0 = wid * (NCHW * 128)
        pltpu.sync_copy(dst_h.at[pl.ds(row0, NCHW)], idxd)
        pltpu.sync_copy(src_h.at[pl.ds(row0, NCHW)], idxs)

        def body(ci, carry):
            pltpu.sync_copy(tabA_h.at[idxd.at[ci]], bufA)
            pltpu.sync_copy(tabB_h.at[idxs.at[ci]], bufB)
            e = e0 + ci * 128
            pltpu.sync_copy(bufA, outA_h.at[pl.ds(e, 128)])
            pltpu.sync_copy(bufB, outB_h.at[pl.ds(e, 128)])
            return carry

        lax.fori_loop(0, NCHW, body, 0)

    return k(tabA, tabB, dst2, src2)


def _scatter(m, dst2, zeros_tab, half):
    RING = 2
    NCHW = ROWS_HA if half == 0 else ROWS_HB

    def k(m_h, dst_h, z_h, out_h, idxd, buf, table, semM):
        c = lax.axis_index("c")
        s = lax.axis_index("s")
        wid = s * NC + c
        row0 = wid * ROWS_W + half * ROWS_HA
        e0 = wid * (NCHW * 128)
        t0 = s * TILE_ROWS
        pltpu.sync_copy(z_h.at[pl.ds(t0, TILE_ROWS)],
                        table.at[pl.ds(t0, TILE_ROWS)])
        pltpu.sync_copy(dst_h.at[pl.ds(row0, NCHW)], idxd)

        def fire(ch, slot):
            pltpu.async_copy(m_h.at[pl.ds(e0 + ch * 128, 128)], buf.at[slot],
                             semM.at[slot])

        for p in range(RING - 1):
            fire(p, p)
        plsc.subcore_barrier()

        def body(ci, carry):
            b = lax.rem(ci, RING)
            pltpu.make_async_copy(m_h.at[pl.ds(e0, 128)], buf.at[b],
                                  semM.at[b]).wait()
            pltpu.sync_copy(buf.at[b], table.at[idxd.at[ci]], add=True)

            @pl.when(ci + RING - 1 <= NCHW - 1)
            def _():
                fire(ci + RING - 1, lax.rem(ci + RING - 1, RING))
            return carry

        lax.fori_loop(0, NCHW, body, 0)
        plsc.subcore_barrier()
        pltpu.sync_copy(table.at[pl.ds(t0, TILE_ROWS)],
                        out_h.at[pl.ds(c * N_PAD + t0, TILE_ROWS)])

    kk = functools.partial(
        pl.kernel,
        out_type=jax.ShapeDtypeStruct((2 * N_PAD, H), f32),
        mesh=_sc_mesh(),
        scratch_types=[
            pltpu.VMEM((NCHW, 128), i32),
            pltpu.VMEM((RING, 128, H), f32),
            pltpu.VMEM_SHARED((N_PAD, H), f32),
            pltpu.SemaphoreType.DMA((RING,)),
        ],
    )(k)
    return kk(m, dst2, zeros_tab)


def _count(dst2, zeros_tab, ones128):
    @functools.partial(
        pl.kernel,
        out_type=jax.ShapeDtypeStruct((2 * N_PAD, H), f32),
        mesh=_sc_mesh(),
        scratch_types=[
            pltpu.VMEM((ROWS_W, 128), i32),
            pltpu.VMEM((128, H), f32),
            pltpu.VMEM_SHARED((N_PAD, H), f32),
        ],
    )
    def k(dst_h, z_h, ones_h, out_h, idx, ones_b, table):
        c = lax.axis_index("c")
        s = lax.axis_index("s")
        wid = s * NC + c
        t0 = s * TILE_ROWS
        pltpu.sync_copy(dst_h.at[pl.ds(wid * ROWS_W, ROWS_W)], idx)
        pltpu.sync_copy(ones_h, ones_b)
        pltpu.sync_copy(z_h.at[pl.ds(t0, TILE_ROWS)],
                        table.at[pl.ds(t0, TILE_ROWS)])
        plsc.subcore_barrier()

        def body(row, carry):
            pltpu.sync_copy(ones_b, table.at[idx.at[row]], add=True)
            return carry

        lax.fori_loop(0, ROWS_W, body, 0)
        plsc.subcore_barrier()
        pltpu.sync_copy(table.at[pl.ds(t0, TILE_ROWS)],
                        out_h.at[pl.ds(c * N_PAD + t0, TILE_ROWS)])

    return k(dst2, zeros_tab, ones128)


def _msplit(lp):
    w = lp['m1']['w']
    return (w[:, :H].T, w[:, H:2 * H].T, w[:, 2 * H:2 * H + TW].T,
            w[:, 2 * H + TW].reshape(1, H), lp['m1']['b'].reshape(1, H))


def _usplit(lp):
    w = lp['u1']['w']
    return (w[:, :H].T, w[:, H:].T, lp['u1']['b'].reshape(1, H),
            lp['u2']['w'].T, lp['u2']['b'].reshape(1, H))


def _dec_mats(params):
    c1w, c1b = params['c1w'], params['c1b']
    c2w, c2b = params['c2w'], params['c2b']
    oo, pp, ii = jnp.meshgrid(jnp.arange(8), jnp.arange(38), jnp.arange(16),
                              indexing='ij')
    M1 = jnp.zeros((H, 8 * 38), f32).at[
        (3 * pp + ii).ravel(), (oo * 38 + pp).ravel()
    ].set(c1w[oo.ravel(), 0, ii.ravel()])
    b1r = jnp.repeat(c1b, 38).reshape(1, 8 * 38)
    o2, tt, jj = jnp.meshgrid(jnp.arange(8), jnp.arange(TW), jnp.arange(14),
                              indexing='ij')
    M2 = jnp.zeros((8 * 38, TW), f32).at[
        (o2 * 38 + tt + jj).ravel(), tt.ravel()
    ].set(c2w[0, o2.ravel(), jj.ravel()])
    b2r = jnp.full((1, TW), c2b[0], f32)
    return M1, b1r, M2, b2r


def kernel(data, _a, _b, dist, edge_index, _c, _d, targets, train_mask, params):
    u = data[:, :, 0]
    u_pad = jnp.zeros((N_PAD, TW), f32).at[:N_NODES].set(u)
    pad_idx = jnp.full((E_PAD - N_EDGES,), N_NODES, i32)
    dst2 = jnp.concatenate([edge_index[1], pad_idx]).reshape(E_PAD // 128, 128)
    src2 = jnp.concatenate([edge_index[0], pad_idx]).reshape(E_PAD // 128, 128)
    dist2 = jnp.concatenate(
        [dist, jnp.zeros((E_PAD - N_EDGES,), f32)]).reshape(E_PAD, 1)
    zeros_tab = jnp.zeros((N_PAD, H), f32)
    ones128 = jnp.full((128, H), 1.0 / H, f32)

    e1 = params['e1']
    e2 = params['e2']
    lps = params['layers']

    C = _count(dst2, zeros_tab, ones128)

    waT, wbT, wuT, wd, b1 = _msplit(lps[0])
    h, A, B = _enc_ab(u_pad, e1['w'].T, e1['b'].reshape(1, H),
                      e2['w'].T, e2['b'].reshape(1, H), waT, wuT, b1, wbT)

    dpw = dist2.reshape(NW, PER_W)
    d2a = dpw[:, :ROWS_HA * 128].reshape(E2A, 1)
    d2b = dpw[:, ROWS_HA * 128:].reshape(E2B, 1)
    for li in range(NLAYERS):
        _, _, _, wd, _ = _msplit(lps[li])
        m2T = lps[li]['m2']['w'].T
        m2b = lps[li]['m2']['b'].reshape(1, H)
        gA1, gB1 = _gather_ab(A, B, dst2, src2, 0)
        gA2, gB2 = _gather_ab(A, B, dst2, src2, 1)
        m1 = _edge(gA1, gB1, d2a, wd, m2T, m2b)
        m2 = _edge(gA2, gB2, d2b, wd, m2T, m2b)
        P1 = _scatter(m1, dst2, zeros_tab, 0)
        P2 = _scatter(m2, dst2, zeros_tab, 1)
        u1hT, u1aT, u1b, u2T, u2b = _usplit(lps[li])
        if li < NLAYERS - 1:
            waT, wbT, wuT, _, b1 = _msplit(lps[li + 1])
            h, A, B = _upd_ab(h, P1, P2, C, u_pad, u1hT, u1aT, u1b, u2T, u2b,
                              waT, wuT, b1, wbT)
        else:
            h = _upd_last(h, P1, P2, C, u1hT, u1aT, u1b, u2T, u2b)

    M1, b1r, M2, b2r = _dec_mats(params)
    o = _dec(h, u_pad, M1, b1r, M2, b2r)
    return o[:N_NODES][:, :, None]


# speedup vs baseline: 1.5088x; 1.5088x over previous
import functools

import jax
import jax.numpy as jnp
from jax import lax
from jax.experimental import pallas as pl
from jax.experimental.pallas import tpu as pltpu
from jax.experimental.pallas import tpu_sc as plsc

N_NODES = 10000
N_EDGES = 160000
TW = 25
H = 128
NLAYERS = 6

N_PAD = 10240
E_PAD = 163840
NC = 2
NS = 16
NW = NC * NS
PER_W = E_PAD // NW
ROWS_W = PER_W // 128
NCH = ROWS_W // 2
NCH2 = ROWS_W
ROWS_HA = 24
ROWS_HB = 16
E2A = ROWS_HA * NW * 128
E2B = ROWS_HB * NW * 128
TILE_ROWS = N_PAD // NS

NBLK = 1024
EBLK = 1024

f32 = jnp.float32
i32 = jnp.int32


def _swish(x):
    return x * jax.nn.sigmoid(x)


def _full(shape):
    return pl.BlockSpec(shape, lambda i: tuple(0 for _ in shape))


def _rows(blk, width, off=0):
    return pl.BlockSpec((blk, width), lambda i, o=off: (i + o, 0))


def _enc_ab_body(u, e1T, e1b, e2T, e2b, waT, wuT, b1, wbT, h_o, a_o, b_o):
    uu = u[...]
    h1 = _swish(jnp.dot(uu, e1T[...], preferred_element_type=f32) + e1b[...])
    h = _swish(jnp.dot(h1, e2T[...], preferred_element_type=f32) + e2b[...])
    uw = jnp.dot(uu, wuT[...], preferred_element_type=f32)
    h_o[...] = h
    a_o[...] = jnp.dot(h, waT[...], preferred_element_type=f32) + uw + b1[...]
    b_o[...] = jnp.dot(h, wbT[...], preferred_element_type=f32) - uw


def _enc_ab(u, e1T, e1b, e2T, e2b, waT, wuT, b1, wbT):
    return pl.pallas_call(
        _enc_ab_body,
        grid=(N_PAD // NBLK,),
        in_specs=[_rows(NBLK, TW), _full((TW, H)), _full((1, H)),
                  _full((H, H)), _full((1, H)), _full((H, H)),
                  _full((TW, H)), _full((1, H)), _full((H, H))],
        out_specs=[_rows(NBLK, H)] * 3,
        out_shape=[jax.ShapeDtypeStruct((N_PAD, H), f32)] * 3,
    )(u, e1T, e1b, e2T, e2b, waT, wuT, b1, wbT)


def _edge_body(ga, gb, dist, wd, m2T, m2b, m_o):
    pre = ga[...] + gb[...] + dist[...] * wd[...]
    m1 = _swish(pre)
    m_o[...] = _swish(jnp.dot(m1, m2T[...], preferred_element_type=f32) + m2b[...])


def _edge(gA, gB, dist2, wd, m2T, m2b):
    eh = gA.shape[0]
    return pl.pallas_call(
        _edge_body,
        grid=(eh // EBLK,),
        in_specs=[_rows(EBLK, H), _rows(EBLK, H), _rows(EBLK, 1),
                  _full((1, H)), _full((H, H)), _full((1, H))],
        out_specs=_rows(EBLK, H),
        out_shape=jax.ShapeDtypeStruct((eh, H), f32),
    )(gA, gB, dist2, wd, m2T, m2b)


def _upd_ab_body(h, p0, p1, q0, q1, c0, c1, u, u1hT, u1aT, u1b, u2T, u2b,
                 waT, wuT, b1, wbT, hn_o, a_o, b_o):
    hh = h[...]
    cnt = jnp.sum(c0[...] + c1[...], axis=1, keepdims=True)
    agg = (p0[...] + p1[...] + q0[...] + q1[...]) / jnp.maximum(cnt, 1.0)
    t = _swish(jnp.dot(hh, u1hT[...], preferred_element_type=f32)
               + jnp.dot(agg, u1aT[...], preferred_element_type=f32) + u1b[...])
    upd = _swish(jnp.dot(t, u2T[...], preferred_element_type=f32) + u2b[...])
    hn = hh + upd
    hn_o[...] = hn
    uw = jnp.dot(u[...], wuT[...], preferred_element_type=f32)
    a_o[...] = jnp.dot(hn, waT[...], preferred_element_type=f32) + uw + b1[...]
    b_o[...] = jnp.dot(hn, wbT[...], preferred_element_type=f32) - uw


def _upd_ab(h, P1, P2, C, u, u1hT, u1aT, u1b, u2T, u2b, waT, wuT, b1, wbT):
    nb = N_PAD // NBLK
    return pl.pallas_call(
        _upd_ab_body,
        grid=(nb,),
        in_specs=[_rows(NBLK, H), _rows(NBLK, H), _rows(NBLK, H, off=nb),
                  _rows(NBLK, H), _rows(NBLK, H, off=nb),
                  _rows(NBLK, H), _rows(NBLK, H, off=nb), _rows(NBLK, TW),
                  _full((H, H)), _full((H, H)), _full((1, H)),
                  _full((H, H)), _full((1, H)),
                  _full((H, H)), _full((TW, H)), _full((1, H)), _full((H, H))],
        out_specs=[_rows(NBLK, H)] * 3,
        out_shape=[jax.ShapeDtypeStruct((N_PAD, H), f32)] * 3,
    )(h, P1, P1, P2, P2, C, C, u, u1hT, u1aT, u1b, u2T, u2b, waT, wuT, b1, wbT)


def _upd_last_body(h, p0, p1, q0, q1, c0, c1, u1hT, u1aT, u1b, u2T, u2b, hn_o):
    hh = h[...]
    cnt = jnp.sum(c0[...] + c1[...], axis=1, keepdims=True)
    agg = (p0[...] + p1[...] + q0[...] + q1[...]) / jnp.maximum(cnt, 1.0)
    t = _swish(jnp.dot(hh, u1hT[...], preferred_element_type=f32)
               + jnp.dot(agg, u1aT[...], preferred_element_type=f32) + u1b[...])
    upd = _swish(jnp.dot(t, u2T[...], preferred_element_type=f32) + u2b[...])
    hn_o[...] = hh + upd


def _upd_last(h, P1, P2, C, u1hT, u1aT, u1b, u2T, u2b):
    nb = N_PAD // NBLK
    return pl.pallas_call(
        _upd_last_body,
        grid=(nb,),
        in_specs=[_rows(NBLK, H), _rows(NBLK, H), _rows(NBLK, H, off=nb),
                  _rows(NBLK, H), _rows(NBLK, H, off=nb),
                  _rows(NBLK, H), _rows(NBLK, H, off=nb),
                  _full((H, H)), _full((H, H)), _full((1, H)),
                  _full((H, H)), _full((1, H))],
        out_specs=_rows(NBLK, H),
        out_shape=jax.ShapeDtypeStruct((N_PAD, H), f32),
    )(h, P1, P1, P2, P2, C, C, u1hT, u1aT, u1b, u2T, u2b)


def _dec_body(h, u, M1, b1r, M2, b2r, o_ref):
    y = _swish(jnp.dot(h[...], M1[...], preferred_element_type=f32) + b1r[...])
    z = jnp.dot(y, M2[...], preferred_element_type=f32) + b2r[...]
    dtv = (lax.broadcasted_iota(i32, (NBLK, TW), 1) + 1).astype(f32)
    o_ref[...] = u[...][:, TW - 1:TW] + dtv * z


def _dec(h, u, M1, b1r, M2, b2r):
    return pl.pallas_call(
        _dec_body,
        grid=(N_PAD // NBLK,),
        in_specs=[_rows(NBLK, H), _rows(NBLK, TW), _full((H, 8 * 38)),
                  _full((1, 8 * 38)), _full((8 * 38, TW)), _full((1, TW))],
        out_specs=_rows(NBLK, TW),
        out_shape=jax.ShapeDtypeStruct((N_PAD, TW), f32),
    )(h, u, M1, b1r, M2, b2r)


def _sc_mesh():
    return plsc.VectorSubcoreMesh(core_axis_name="c", subcore_axis_name="s")


def _gather_ab(tabA, tabB, dst2, src2, half):
    RING = 3
    NCHW = ROWS_HA if half == 0 else ROWS_HB
    EH = NCHW * NW * 128

    @functools.partial(
        pl.kernel,
        out_type=(jax.ShapeDtypeStruct((EH, H), f32),
                  jax.ShapeDtypeStruct((EH, H), f32)),
        mesh=_sc_mesh(),
        scratch_types=[
            pltpu.VMEM((NCHW, 128), i32), pltpu.VMEM((NCHW, 128), i32),
            pltpu.VMEM((RING, 128, H), f32), pltpu.VMEM((RING, 128, H), f32),
            pltpu.SemaphoreType.DMA((RING,)), pltpu.SemaphoreType.DMA((RING,)),
            pltpu.SemaphoreType.DMA((RING,)), pltpu.SemaphoreType.DMA((RING,)),
        ],
    )
    def k(tabA_h, tabB_h, dst_h, src_h, outA_h, outB_h,
          idxd, idxs, bufA, bufB, semA, semB, semWA, semWB):
        c = lax.axis_index("c")
        s = lax.axis_index("s")
        wid = s * NC + c
        row0 = wid * ROWS_W + half * ROWS_HA
        e0 = wid * (NCHW * 128)
        pltpu.sync_copy(dst_h.at[pl.ds(row0, NCHW)], idxd)
        pltpu.sync_copy(src_h.at[pl.ds(row0, NCHW)], idxs)

        def fire(ch, slot):
            pltpu.async_copy(tabA_h.at[idxd.at[ch]], bufA.at[slot],
                             semA.at[slot])
            pltpu.async_copy(tabB_h.at[idxs.at[ch]], bufB.at[slot],
                             semB.at[slot])

        for p in range(RING - 1):
            fire(p, p)

        def body(ci, carry):
            b = lax.rem(ci, RING)
            pltpu.make_async_copy(tabA_h.at[idxd.at[ci]], bufA.at[b],
                                  semA.at[b]).wait()
            pltpu.make_async_copy(tabB_h.at[idxs.at[ci]], bufB.at[b],
                                  semB.at[b]).wait()
            e = e0 + ci * 128
            pltpu.async_copy(bufA.at[b], outA_h.at[pl.ds(e, 128)],
                             semWA.at[b])
            pltpu.async_copy(bufB.at[b], outB_h.at[pl.ds(e, 128)],
                             semWB.at[b])

            @pl.when(ci + RING - 1 <= NCHW - 1)
            def _():
                b2 = lax.rem(ci + RING - 1, RING)

                @pl.when(ci >= 1)
                def _():
                    pltpu.make_async_copy(bufA.at[b2],
                                          outA_h.at[pl.ds(e, 128)],
                                          semWA.at[b2]).wait()
                    pltpu.make_async_copy(bufB.at[b2],
                                          outB_h.at[pl.ds(e, 128)],
                                          semWB.at[b2]).wait()
                fire(ci + RING - 1, b2)
            return carry

        lax.fori_loop(0, NCHW, body, 0)
        for p in range(RING):
            pltpu.make_async_copy(bufA.at[p], outA_h.at[pl.ds(e0, 128)],
                                  semWA.at[p]).wait()
            pltpu.make_async_copy(bufB.at[p], outB_h.at[pl.ds(e0, 128)],
                                  semWB.at[p]).wait()

    return k(tabA, tabB, dst2, src2)


def _scatter(m, dst2, zeros_tab, half):
    RING = 2
    NCHW = ROWS_HA if half == 0 else ROWS_HB

    def k(m_h, dst_h, z_h, out_h, idxd, buf, table, semM):
        c = lax.axis_index("c")
        s = lax.axis_index("s")
        wid = s * NC + c
        row0 = wid * ROWS_W + half * ROWS_HA
        e0 = wid * (NCHW * 128)
        t0 = s * TILE_ROWS
        pltpu.sync_copy(z_h.at[pl.ds(t0, TILE_ROWS)],
                        table.at[pl.ds(t0, TILE_ROWS)])
        pltpu.sync_copy(dst_h.at[pl.ds(row0, NCHW)], idxd)

        def fire(ch, slot):
            pltpu.async_copy(m_h.at[pl.ds(e0 + ch * 128, 128)], buf.at[slot],
                             semM.at[slot])

        for p in range(RING - 1):
            fire(p, p)
        plsc.subcore_barrier()

        def body(ci, carry):
            b = lax.rem(ci, RING)
            pltpu.make_async_copy(m_h.at[pl.ds(e0, 128)], buf.at[b],
                                  semM.at[b]).wait()
            pltpu.sync_copy(buf.at[b], table.at[idxd.at[ci]], add=True)

            @pl.when(ci + RING - 1 <= NCHW - 1)
            def _():
                fire(ci + RING - 1, lax.rem(ci + RING - 1, RING))
            return carry

        lax.fori_loop(0, NCHW, body, 0)
        plsc.subcore_barrier()
        pltpu.sync_copy(table.at[pl.ds(t0, TILE_ROWS)],
                        out_h.at[pl.ds(c * N_PAD + t0, TILE_ROWS)])

    kk = functools.partial(
        pl.kernel,
        out_type=jax.ShapeDtypeStruct((2 * N_PAD, H), f32),
        mesh=_sc_mesh(),
        scratch_types=[
            pltpu.VMEM((NCHW, 128), i32),
            pltpu.VMEM((RING, 128, H), f32),
            pltpu.VMEM_SHARED((N_PAD, H), f32),
            pltpu.SemaphoreType.DMA((RING,)),
        ],
    )(k)
    return kk(m, dst2, zeros_tab)


def _count(dst2, zeros_tab, ones128):
    @functools.partial(
        pl.kernel,
        out_type=jax.ShapeDtypeStruct((2 * N_PAD, H), f32),
        mesh=_sc_mesh(),
        scratch_types=[
            pltpu.VMEM((ROWS_W, 128), i32),
            pltpu.VMEM((128, H), f32),
            pltpu.VMEM_SHARED((N_PAD, H), f32),
        ],
    )
    def k(dst_h, z_h, ones_h, out_h, idx, ones_b, table):
        c = lax.axis_index("c")
        s = lax.axis_index("s")
        wid = s * NC + c
        t0 = s * TILE_ROWS
        pltpu.sync_copy(dst_h.at[pl.ds(wid * ROWS_W, ROWS_W)], idx)
        pltpu.sync_copy(ones_h, ones_b)
        pltpu.sync_copy(z_h.at[pl.ds(t0, TILE_ROWS)],
                        table.at[pl.ds(t0, TILE_ROWS)])
        plsc.subcore_barrier()

        def body(row, carry):
            pltpu.sync_copy(ones_b, table.at[idx.at[row]], add=True)
            return carry

        lax.fori_loop(0, ROWS_W, body, 0)
        plsc.subcore_barrier()
        pltpu.sync_copy(table.at[pl.ds(t0, TILE_ROWS)],
                        out_h.at[pl.ds(c * N_PAD + t0, TILE_ROWS)])

    return k(dst2, zeros_tab, ones128)


def _msplit(lp):
    w = lp['m1']['w']
    return (w[:, :H].T, w[:, H:2 * H].T, w[:, 2 * H:2 * H + TW].T,
            w[:, 2 * H + TW].reshape(1, H), lp['m1']['b'].reshape(1, H))


def _usplit(lp):
    w = lp['u1']['w']
    return (w[:, :H].T, w[:, H:].T, lp['u1']['b'].reshape(1, H),
            lp['u2']['w'].T, lp['u2']['b'].reshape(1, H))


def _dec_mats(params):
    c1w, c1b = params['c1w'], params['c1b']
    c2w, c2b = params['c2w'], params['c2b']
    oo, pp, ii = jnp.meshgrid(jnp.arange(8), jnp.arange(38), jnp.arange(16),
                              indexing='ij')
    M1 = jnp.zeros((H, 8 * 38), f32).at[
        (3 * pp + ii).ravel(), (oo * 38 + pp).ravel()
    ].set(c1w[oo.ravel(), 0, ii.ravel()])
    b1r = jnp.repeat(c1b, 38).reshape(1, 8 * 38)
    o2, tt, jj = jnp.meshgrid(jnp.arange(8), jnp.arange(TW), jnp.arange(14),
                              indexing='ij')
    M2 = jnp.zeros((8 * 38, TW), f32).at[
        (o2 * 38 + tt + jj).ravel(), tt.ravel()
    ].set(c2w[0, o2.ravel(), jj.ravel()])
    b2r = jnp.full((1, TW), c2b[0], f32)
    return M1, b1r, M2, b2r


def kernel(data, _a, _b, dist, edge_index, _c, _d, targets, train_mask, params):
    u = data[:, :, 0]
    u_pad = jnp.zeros((N_PAD, TW), f32).at[:N_NODES].set(u)
    pad_idx = jnp.full((E_PAD - N_EDGES,), N_NODES, i32)
    dst2 = jnp.concatenate([edge_index[1], pad_idx]).reshape(E_PAD // 128, 128)
    src2 = jnp.concatenate([edge_index[0], pad_idx]).reshape(E_PAD // 128, 128)
    dist2 = jnp.concatenate(
        [dist, jnp.zeros((E_PAD - N_EDGES,), f32)]).reshape(E_PAD, 1)
    zeros_tab = jnp.zeros((N_PAD, H), f32)
    ones128 = jnp.full((128, H), 1.0 / H, f32)

    e1 = params['e1']
    e2 = params['e2']
    lps = params['layers']

    C = _count(dst2, zeros_tab, ones128)

    waT, wbT, wuT, wd, b1 = _msplit(lps[0])
    h, A, B = _enc_ab(u_pad, e1['w'].T, e1['b'].reshape(1, H),
                      e2['w'].T, e2['b'].reshape(1, H), waT, wuT, b1, wbT)

    dpw = dist2.reshape(NW, PER_W)
    d2a = dpw[:, :ROWS_HA * 128].reshape(E2A, 1)
    d2b = dpw[:, ROWS_HA * 128:].reshape(E2B, 1)
    for li in range(NLAYERS):
        _, _, _, wd, _ = _msplit(lps[li])
        m2T = lps[li]['m2']['w'].T
        m2b = lps[li]['m2']['b'].reshape(1, H)
        gA1, gB1 = _gather_ab(A, B, dst2, src2, 0)
        gA2, gB2 = _gather_ab(A, B, dst2, src2, 1)
        m1 = _edge(gA1, gB1, d2a, wd, m2T, m2b)
        m2 = _edge(gA2, gB2, d2b, wd, m2T, m2b)
        P1 = _scatter(m1, dst2, zeros_tab, 0)
        P2 = _scatter(m2, dst2, zeros_tab, 1)
        u1hT, u1aT, u1b, u2T, u2b = _usplit(lps[li])
        if li < NLAYERS - 1:
            waT, wbT, wuT, _, b1 = _msplit(lps[li + 1])
            h, A, B = _upd_ab(h, P1, P2, C, u_pad, u1hT, u1aT, u1b, u2T, u2b,
                              waT, wuT, b1, wbT)
        else:
            h = _upd_last(h, P1, P2, C, u1hT, u1aT, u1b, u2T, u2b)

    M1, b1r, M2, b2r = _dec_mats(params)
    o = _dec(h, u_pad, M1, b1r, M2, b2r)
    return o[:N_NODES][:, :, None]
